# baseline (speedup 1.0000x reference)
import functools

import jax
import jax.numpy as jnp
from jax import lax
from jax.experimental import pallas as pl
from jax.experimental.pallas import tpu as pltpu
from jax.experimental.pallas import tpu_sc as plsc

NEG = -3.0e38
IBIG = 2**31 - 1


def _make_topk_call(mode, q_total, qb, d, kbook, tile, topk, interpret=False):
    kt = (kbook + tile - 1) // tile

    def body(z_ref, bt_ref, vals_out, idx_out, rv, ri, sm):
        j = pl.program_id(1)

        @pl.when(j == 0)
        def _init():
            rv[...] = jnp.full((qb, topk), NEG, jnp.float32)
            ri[...] = (
                lax.broadcasted_iota(jnp.int32, (qb, topk), 1)
                + jnp.int32(0x40000000)
            )

        z = z_ref[...]
        bt = bt_ref[...]
        if mode == "cos":
            nrm = jnp.sqrt(jnp.sum(z * z, axis=1, keepdims=True))
            q = z / nrm
            s = lax.dot_general(
                q, bt, (((1,), (0,)), ((), ())),
                preferred_element_type=jnp.float32,
            )
        else:
            zz = jnp.sum(z * z, axis=1, keepdims=True)
            bb = jnp.sum(bt * bt, axis=0, keepdims=True)
            s = lax.dot_general(
                z, bt, (((1,), (0,)), ((), ())),
                preferred_element_type=jnp.float32,
            )
            s = 2.0 * s - zz - bb
        col = j * tile + lax.broadcasted_iota(jnp.int32, (1, tile), 1)
        s = jnp.where(col < kbook, s, NEG)
        sm[...] = s
        iota = lax.broadcasted_iota(jnp.int32, (qb, topk), 1)

        def cond(c):
            m, thr = c
            return jnp.any(m > thr)

        def loop_body(c):
            m, thr = c
            s_cur = sm[...]
            sel = jnp.min(jnp.where(s_cur == m, col, IBIG), axis=1,
                          keepdims=True)
            run_v = rv[...]
            run_i = ri[...]
            upd = m > thr
            pos = jnp.sum((run_v >= m).astype(jnp.int32), axis=1,
                          keepdims=True)
            pos = jnp.where(upd, pos, topk)
            sh_v = jnp.concatenate([run_v[:, :1], run_v[:, :topk - 1]],
                                   axis=1)
            sh_i = jnp.concatenate([run_i[:, :1], run_i[:, :topk - 1]],
                                   axis=1)
            nv = jnp.where(iota < pos, run_v,
                           jnp.where(iota == pos, m, sh_v))
            ni = jnp.where(iota < pos, run_i,
                           jnp.where(iota == pos, sel, sh_i))
            rv[...] = nv
            ri[...] = ni
            s_new = jnp.where(col == sel, NEG, s_cur)
            sm[...] = s_new
            m_new = jnp.max(s_new, axis=1, keepdims=True)
            return m_new, nv[:, topk - 1:topk]

        m0 = jnp.max(s, axis=1, keepdims=True)
        thr0 = rv[:, topk - 1:topk]
        lax.while_loop(cond, loop_body, (m0, thr0))

        @pl.when(j == kt - 1)
        def _flush():
            vals_out[...] = rv[...]
            idx_out[...] = ri[...]

    return pl.pallas_call(
        body,
        grid=(q_total // qb, kt),
        in_specs=[
            pl.BlockSpec((qb, d), lambda i, j: (i, 0)),
            pl.BlockSpec((d, tile), lambda i, j: (0, j)),
        ],
        out_specs=[
            pl.BlockSpec((qb, topk), lambda i, j: (i, 0)),
            pl.BlockSpec((qb, topk), lambda i, j: (i, 0)),
        ],
        out_shape=[
            jax.ShapeDtypeStruct((q_total, topk), jnp.float32),
            jax.ShapeDtypeStruct((q_total, topk), jnp.int32),
        ],
        scratch_shapes=[
            pltpu.VMEM((qb, topk), jnp.float32),
            pltpu.VMEM((qb, topk), jnp.int32),
            pltpu.VMEM((qb, tile), jnp.float32),
        ],
        compiler_params=pltpu.CompilerParams(
            dimension_semantics=("parallel", "arbitrary"),
        ),
        interpret=interpret,
    )


Q, D, KBOOK, L, TOPK = 1024, 32, 100000, 3, 20
QB, TILE = 256, 2048

_cos_call = _make_topk_call("cos", Q, QB, D, KBOOK, TILE, TOPK)
_euc_call = _make_topk_call("euc", Q, QB, D, KBOOK, TILE, TOPK)

_SC_NC, _SC_NS = 2, 16
_NW = _SC_NC * _SC_NS
_B = Q * TOPK
_BPW = _B // _NW
_LPAD = 128

_gather_mesh = plsc.VectorSubcoreMesh(core_axis_name="c", subcore_axis_name="s")


@functools.partial(
    pl.kernel,
    mesh=_gather_mesh,
    out_type=jax.ShapeDtypeStruct((_B, _LPAD), jnp.float32),
    scratch_types=[
        pltpu.VMEM((_BPW,), jnp.int32),
        pltpu.VMEM((_BPW, _LPAD), jnp.float32),
        pltpu.SemaphoreType.DMA,
    ],
)
def _label_gather(table_hbm, idx_hbm, out_hbm, idx_v, rows_v, sem):
    wid = lax.axis_index("s") * _SC_NC + lax.axis_index("c")
    base = wid * _BPW
    pltpu.sync_copy(idx_hbm.at[pl.ds(base, _BPW)], idx_v)
    pltpu.async_copy(table_hbm.at[idx_v], rows_v, sem).wait()
    pltpu.sync_copy(rows_v, out_hbm.at[pl.ds(base, _BPW)])


def _labels(label_book, ind):
    pad = jnp.pad(label_book, ((0, 0), (0, _LPAD - L)))
    out = _label_gather(pad, ind.reshape(-1))
    return out[:, :L].reshape(Q, TOPK, L)


@jax.jit
def kernel(z_rot, z_trans, z_rot_book, z_trans_book, rot_book, trans_book, k):
    rot_vals, rot_ind = _cos_call(z_rot, z_rot_book.T)
    trans_vals, trans_ind = _euc_call(z_trans, z_trans_book.T)
    rot_labels = _labels(rot_book, rot_ind)
    trans_labels = _labels(trans_book, trans_ind)
    return (rot_vals, rot_ind, rot_labels, trans_vals, trans_ind, trans_labels)


# speedup vs baseline: 1.1170x; 1.1170x over previous
import functools

import jax
import jax.numpy as jnp
from jax import lax
from jax.experimental import pallas as pl
from jax.experimental.pallas import tpu as pltpu
from jax.experimental.pallas import tpu_sc as plsc

NEG = -3.0e38
IBIG = 2**31 - 1


def _make_topk_call(mode, q_total, qb, d, kbook, tile, topk, interpret=False):
    kt = (kbook + tile - 1) // tile

    def body(z_ref, bt_ref, vals_out, idx_out, rv, ri, sm):
        j = pl.program_id(1)

        @pl.when(j == 0)
        def _init():
            rv[...] = jnp.full((qb, topk), NEG, jnp.float32)
            ri[...] = (
                lax.broadcasted_iota(jnp.int32, (qb, topk), 1)
                + jnp.int32(0x40000000)
            )

        z = z_ref[...]
        bt = bt_ref[...]
        if mode == "cos":
            nrm = jnp.sqrt(jnp.sum(z * z, axis=1, keepdims=True))
            q = z / nrm
            s = lax.dot_general(
                q, bt, (((1,), (0,)), ((), ())),
                preferred_element_type=jnp.float32,
            )
        else:
            zz = jnp.sum(z * z, axis=1, keepdims=True)
            bb = jnp.sum(bt * bt, axis=0, keepdims=True)
            s = lax.dot_general(
                z, bt, (((1,), (0,)), ((), ())),
                preferred_element_type=jnp.float32,
            )
            s = 2.0 * s - zz - bb
        col = j * tile + lax.broadcasted_iota(jnp.int32, (1, tile), 1)
        s = jnp.where(col < kbook, s, NEG)
        sm[...] = s
        iota = lax.broadcasted_iota(jnp.int32, (qb, topk), 1)

        def cond(c):
            m, thr = c
            return jnp.any(m > thr)

        def loop_body(c):
            m, thr = c
            s_cur = sm[...]
            sel = jnp.min(jnp.where(s_cur == m, col, IBIG), axis=1,
                          keepdims=True)
            run_v = rv[...]
            run_i = ri[...]
            upd = m > thr
            pos = jnp.sum((run_v >= m).astype(jnp.int32), axis=1,
                          keepdims=True)
            pos = jnp.where(upd, pos, topk)
            sh_v = jnp.concatenate([run_v[:, :1], run_v[:, :topk - 1]],
                                   axis=1)
            sh_i = jnp.concatenate([run_i[:, :1], run_i[:, :topk - 1]],
                                   axis=1)
            nv = jnp.where(iota < pos, run_v,
                           jnp.where(iota == pos, m, sh_v))
            ni = jnp.where(iota < pos, run_i,
                           jnp.where(iota == pos, sel, sh_i))
            rv[...] = nv
            ri[...] = ni
            s_new = jnp.where(col == sel, NEG, s_cur)
            sm[...] = s_new
            m_new = jnp.max(s_new, axis=1, keepdims=True)
            return m_new, nv[:, topk - 1:topk]

        m0 = jnp.max(s, axis=1, keepdims=True)
        thr0 = rv[:, topk - 1:topk]
        lax.while_loop(cond, loop_body, (m0, thr0))

        @pl.when(j == kt - 1)
        def _flush():
            vals_out[...] = rv[...]
            idx_out[...] = ri[...]

    return pl.pallas_call(
        body,
        grid=(q_total // qb, kt),
        in_specs=[
            pl.BlockSpec((qb, d), lambda i, j: (i, 0)),
            pl.BlockSpec((d, tile), lambda i, j: (0, j)),
        ],
        out_specs=[
            pl.BlockSpec((qb, topk), lambda i, j: (i, 0)),
            pl.BlockSpec((qb, topk), lambda i, j: (i, 0)),
        ],
        out_shape=[
            jax.ShapeDtypeStruct((q_total, topk), jnp.float32),
            jax.ShapeDtypeStruct((q_total, topk), jnp.int32),
        ],
        scratch_shapes=[
            pltpu.VMEM((qb, topk), jnp.float32),
            pltpu.VMEM((qb, topk), jnp.int32),
            pltpu.VMEM((qb, tile), jnp.float32),
        ],
        compiler_params=pltpu.CompilerParams(
            dimension_semantics=("parallel", "arbitrary"),
        ),
        interpret=interpret,
    )


Q, D, KBOOK, L, TOPK = 1024, 32, 100000, 3, 20
QB, TILE = 1024, 2048

_cos_call = _make_topk_call("cos", Q, QB, D, KBOOK, TILE, TOPK)
_euc_call = _make_topk_call("euc", Q, QB, D, KBOOK, TILE, TOPK)

_SC_NC, _SC_NS = 2, 16
_NW = _SC_NC * _SC_NS
_B = Q * TOPK
_BPW = _B // _NW
_LPAD = 128

_gather_mesh = plsc.VectorSubcoreMesh(core_axis_name="c", subcore_axis_name="s")


@functools.partial(
    pl.kernel,
    mesh=_gather_mesh,
    out_type=jax.ShapeDtypeStruct((_B, _LPAD), jnp.float32),
    scratch_types=[
        pltpu.VMEM((_BPW,), jnp.int32),
        pltpu.VMEM((_BPW, _LPAD), jnp.float32),
        pltpu.SemaphoreType.DMA,
    ],
)
def _label_gather(table_hbm, idx_hbm, out_hbm, idx_v, rows_v, sem):
    wid = lax.axis_index("s") * _SC_NC + lax.axis_index("c")
    base = wid * _BPW
    pltpu.sync_copy(idx_hbm.at[pl.ds(base, _BPW)], idx_v)
    pltpu.async_copy(table_hbm.at[idx_v], rows_v, sem).wait()
    pltpu.sync_copy(rows_v, out_hbm.at[pl.ds(base, _BPW)])


def _labels(label_book, ind):
    pad = jnp.pad(label_book, ((0, 0), (0, _LPAD - L)))
    out = _label_gather(pad, ind.reshape(-1))
    return out[:, :L].reshape(Q, TOPK, L)


@jax.jit
def kernel(z_rot, z_trans, z_rot_book, z_trans_book, rot_book, trans_book, k):
    rot_vals, rot_ind = _cos_call(z_rot, z_rot_book.T)
    trans_vals, trans_ind = _euc_call(z_trans, z_trans_book.T)
    rot_labels = _labels(rot_book, rot_ind)
    trans_labels = _labels(trans_book, trans_ind)
    return (rot_vals, rot_ind, rot_labels, trans_vals, trans_ind, trans_labels)
